# Initial kernel scaffold; baseline (speedup 1.0000x reference)
import jax
import jax.numpy as jnp
from jax.experimental import pallas as pl


def kernel(user_indices, item_indices, user_tables, item_tables, user_W1, user_b1, user_W2, user_b2, item_W1, item_b1, item_W2, item_b2):
    raise NotImplementedError("write your pallas kernel here")


# speedup vs baseline: 8.0866x; 8.0866x over previous
import functools

import jax
import jax.numpy as jnp
from jax import lax
from jax.experimental import pallas as pl
from jax.experimental.pallas import tpu as pltpu
from jax.experimental.pallas import tpu_sc as plsc

N_FIELD = 13
VOCAB = 100000
EMB = 16
B = 16384
CONCAT = N_FIELD * EMB
H1, H2 = 64, 32

NC, NS = 2, 16
NW = NC * NS
BPW = B // NW
SUB = 128
NSUB = BPW // SUB


def _sc_gather_body(ut_hbm, it_hbm, uidx_hbm, iidx_hbm, xu_hbm, xi_hbm,
                    idx_v, rows_v, sem):
    wid = lax.axis_index("s") * NC + lax.axis_index("c")
    base = wid * BPW
    for tbl, idx_hbm, out_hbm in ((ut_hbm, uidx_hbm, xu_hbm),
                                  (it_hbm, iidx_hbm, xi_hbm)):
        for f in range(N_FIELD):
            pltpu.sync_copy(idx_hbm.at[f, pl.ds(base, BPW)], idx_v)
            off = jnp.int32(f * VOCAB)

            def _add(j, _, off=off):
                sl = pl.ds(j * 16, 16)
                idx_v[sl] = idx_v[sl] + off
                return 0

            lax.fori_loop(0, BPW // 16, _add, 0)
            cps = [pltpu.async_copy(tbl.at[idx_v.at[pl.ds(k * SUB, SUB)]],
                                    rows_v.at[pl.ds(k * SUB, SUB)], sem)
                   for k in range(NSUB)]
            for c in cps:
                c.wait()
            pltpu.sync_copy(rows_v,
                            out_hbm.at[pl.ds(base, BPW), pl.ds(f * EMB, EMB)])


_sc_gather = functools.partial(
    pl.kernel,
    out_type=(jax.ShapeDtypeStruct((B, CONCAT), jnp.float32),
              jax.ShapeDtypeStruct((B, CONCAT), jnp.float32)),
    mesh=plsc.VectorSubcoreMesh(core_axis_name="c", subcore_axis_name="s"),
    scratch_types=[pltpu.VMEM((BPW,), jnp.int32),
                   pltpu.VMEM((BPW, EMB), jnp.float32),
                   pltpu.SemaphoreType.DMA],
    compiler_params=pltpu.CompilerParams(use_tc_tiling_on_sc=False),
)(_sc_gather_body)


GRID = 8
TB = B // GRID


def _tc_dense_body(xu_ref, xi_ref, uw1, ub1, uw2, ub2, iw1, ib1, iw2, ib2,
                   out_ref, acc):
    hu = jnp.maximum(
        jnp.dot(xu_ref[...], uw1[...], preferred_element_type=jnp.float32)
        + ub1[...], 0.0)
    hu = jnp.maximum(
        jnp.dot(hu, uw2[...], preferred_element_type=jnp.float32)
        + ub2[...], 0.0)
    hi = jnp.maximum(
        jnp.dot(xi_ref[...], iw1[...], preferred_element_type=jnp.float32)
        + ib1[...], 0.0)
    hi = jnp.maximum(
        jnp.dot(hi, iw2[...], preferred_element_type=jnp.float32)
        + ib2[...], 0.0)
    pdot = jnp.sum(hu * hi)
    pnu = jnp.sum(hu * hu)
    pni = jnp.sum(hi * hi)
    i = pl.program_id(0)

    @pl.when(i == 0)
    def _():
        acc[0] = pdot
        acc[1] = pnu
        acc[2] = pni

    @pl.when(i > 0)
    def _():
        acc[0] += pdot
        acc[1] += pnu
        acc[2] += pni

    @pl.when(i == pl.num_programs(0) - 1)
    def _():
        cos = acc[0] / (jnp.sqrt(acc[1]) * jnp.sqrt(acc[2]))
        out_ref[...] = jnp.full((1, 1), jax.nn.sigmoid(cos), jnp.float32)


_tc_dense = pl.pallas_call(
    _tc_dense_body,
    grid=(GRID,),
    in_specs=[
        pl.BlockSpec((TB, CONCAT), lambda i: (i, 0)),
        pl.BlockSpec((TB, CONCAT), lambda i: (i, 0)),
        pl.BlockSpec((CONCAT, H1), lambda i: (0, 0)),
        pl.BlockSpec((1, H1), lambda i: (0, 0)),
        pl.BlockSpec((H1, H2), lambda i: (0, 0)),
        pl.BlockSpec((1, H2), lambda i: (0, 0)),
        pl.BlockSpec((CONCAT, H1), lambda i: (0, 0)),
        pl.BlockSpec((1, H1), lambda i: (0, 0)),
        pl.BlockSpec((H1, H2), lambda i: (0, 0)),
        pl.BlockSpec((1, H2), lambda i: (0, 0)),
    ],
    out_specs=pl.BlockSpec((1, 1), lambda i: (0, 0)),
    out_shape=jax.ShapeDtypeStruct((1, 1), jnp.float32),
    scratch_shapes=[pltpu.SMEM((3,), jnp.float32)],
)


def kernel(user_indices, item_indices, user_tables, item_tables,
           user_W1, user_b1, user_W2, user_b2,
           item_W1, item_b1, item_W2, item_b2):
    ut = user_tables.reshape(N_FIELD * VOCAB, EMB)
    it = item_tables.reshape(N_FIELD * VOCAB, EMB)
    xu, xi = _sc_gather(ut, it, user_indices, item_indices)
    return _tc_dense(xu, xi,
                     user_W1, user_b1.reshape(1, H1),
                     user_W2, user_b2.reshape(1, H2),
                     item_W1, item_b1.reshape(1, H1),
                     item_W2, item_b2.reshape(1, H2))


# speedup vs baseline: 8.1043x; 1.0022x over previous
import functools

import jax
import jax.numpy as jnp
from jax import lax
from jax.experimental import pallas as pl
from jax.experimental.pallas import tpu as pltpu
from jax.experimental.pallas import tpu_sc as plsc

N_FIELD = 13
VOCAB = 100000
EMB = 16
B = 16384
CONCAT = N_FIELD * EMB
H1, H2 = 64, 32

NC, NS = 2, 16
NW = NC * NS
BPW = B // NW
SUB = 128
NSUB = BPW // SUB


def _sc_gather_body(ut_hbm, it_hbm, uidx_hbm, iidx_hbm, xu_hbm, xi_hbm,
                    idx_v, rows_v, sem):
    wid = lax.axis_index("s") * NC + lax.axis_index("c")
    base = wid * BPW
    for tbl, idx_hbm, out_hbm in ((ut_hbm, uidx_hbm, xu_hbm),
                                  (it_hbm, iidx_hbm, xi_hbm)):
        for f in range(N_FIELD):
            pltpu.sync_copy(idx_hbm.at[f, pl.ds(base, BPW)], idx_v)
            cps = [pltpu.async_copy(tbl.at[f].at[idx_v.at[pl.ds(k * SUB, SUB)]],
                                    rows_v.at[pl.ds(k * SUB, SUB)], sem)
                   for k in range(NSUB)]
            for c in cps:
                c.wait()
            pltpu.sync_copy(rows_v,
                            out_hbm.at[pl.ds(base, BPW), pl.ds(f * EMB, EMB)])


_sc_gather = functools.partial(
    pl.kernel,
    out_type=(jax.ShapeDtypeStruct((B, CONCAT), jnp.float32),
              jax.ShapeDtypeStruct((B, CONCAT), jnp.float32)),
    mesh=plsc.VectorSubcoreMesh(core_axis_name="c", subcore_axis_name="s"),
    scratch_types=[pltpu.VMEM((BPW,), jnp.int32),
                   pltpu.VMEM((BPW, EMB), jnp.float32),
                   pltpu.SemaphoreType.DMA],
    compiler_params=pltpu.CompilerParams(use_tc_tiling_on_sc=False),
)(_sc_gather_body)


GRID = 8
TB = B // GRID


def _tc_dense_body(xu_ref, xi_ref, uw1, ub1, uw2, ub2, iw1, ib1, iw2, ib2,
                   out_ref, acc):
    hu = jnp.maximum(
        jnp.dot(xu_ref[...], uw1[...], preferred_element_type=jnp.float32)
        + ub1[...], 0.0)
    hu = jnp.maximum(
        jnp.dot(hu, uw2[...], preferred_element_type=jnp.float32)
        + ub2[...], 0.0)
    hi = jnp.maximum(
        jnp.dot(xi_ref[...], iw1[...], preferred_element_type=jnp.float32)
        + ib1[...], 0.0)
    hi = jnp.maximum(
        jnp.dot(hi, iw2[...], preferred_element_type=jnp.float32)
        + ib2[...], 0.0)
    pdot = jnp.sum(hu * hi)
    pnu = jnp.sum(hu * hu)
    pni = jnp.sum(hi * hi)
    i = pl.program_id(0)

    @pl.when(i == 0)
    def _():
        acc[0] = pdot
        acc[1] = pnu
        acc[2] = pni

    @pl.when(i > 0)
    def _():
        acc[0] += pdot
        acc[1] += pnu
        acc[2] += pni

    @pl.when(i == pl.num_programs(0) - 1)
    def _():
        cos = acc[0] / (jnp.sqrt(acc[1]) * jnp.sqrt(acc[2]))
        out_ref[...] = jnp.full((1, 1), jax.nn.sigmoid(cos), jnp.float32)


_tc_dense = pl.pallas_call(
    _tc_dense_body,
    grid=(GRID,),
    in_specs=[
        pl.BlockSpec((TB, CONCAT), lambda i: (i, 0)),
        pl.BlockSpec((TB, CONCAT), lambda i: (i, 0)),
        pl.BlockSpec((CONCAT, H1), lambda i: (0, 0)),
        pl.BlockSpec((1, H1), lambda i: (0, 0)),
        pl.BlockSpec((H1, H2), lambda i: (0, 0)),
        pl.BlockSpec((1, H2), lambda i: (0, 0)),
        pl.BlockSpec((CONCAT, H1), lambda i: (0, 0)),
        pl.BlockSpec((1, H1), lambda i: (0, 0)),
        pl.BlockSpec((H1, H2), lambda i: (0, 0)),
        pl.BlockSpec((1, H2), lambda i: (0, 0)),
    ],
    out_specs=pl.BlockSpec((1, 1), lambda i: (0, 0)),
    out_shape=jax.ShapeDtypeStruct((1, 1), jnp.float32),
    scratch_shapes=[pltpu.SMEM((3,), jnp.float32)],
)


def kernel(user_indices, item_indices, user_tables, item_tables,
           user_W1, user_b1, user_W2, user_b2,
           item_W1, item_b1, item_W2, item_b2):
    xu, xi = _sc_gather(user_tables, item_tables, user_indices, item_indices)
    return _tc_dense(xu, xi,
                     user_W1, user_b1.reshape(1, H1),
                     user_W2, user_b2.reshape(1, H2),
                     item_W1, item_b1.reshape(1, H1),
                     item_W2, item_b2.reshape(1, H2))
